# baseline (speedup 1.0000x reference)
import jax
import jax.numpy as jnp
from jax import lax
from jax.experimental import pallas as pl
from jax.experimental.pallas import tpu as pltpu
from jax.experimental.pallas import tpu_sc as plsc

N = 10000
E = 160000
D = 256
DQ = 64
NQ = 4
NC = 2
NS = 16
CHUNK = 128
CPT = 80
EPT = CPT * CHUNK
K = 5
ACC_ROWS = 10240


def _sc_body(xp4, xa4, s_c, d_c, s_w, d_w, s_wb, d_wb, zeros_h, zeros16_h,
             ones_h,
             agg_c, cnt_c, agg_w, cnt_w, agg_wb, cnt_wb,
             src_v, dst_v, g0, ones_v, zer_v, cbuf,
             acc_sh, cnt_sh, gsem, ssem, osem):
    cid = lax.axis_index("c")
    sid = lax.axis_index("s")
    pltpu.sync_copy(ones_h, ones_v)
    pltpu.sync_copy(zeros_h, zer_v)

    for table, s4, d3, agg_o, cnt_o in (
        (xp4, s_c, d_c, agg_c, cnt_c),
        (xa4, s_w, d_w, agg_w, cnt_w),
        (xp4, s_wb, d_wb, agg_wb, cnt_wb),
    ):
        pltpu.sync_copy(d3.at[sid], dst_v)
        for p in range(2):
            q = 2 * p + cid
            do_cnt = p == 0
            for k in range(5):
                pltpu.sync_copy(zer_v, acc_sh.at[pl.ds(sid * 640 + k * 128, 128)])
            if do_cnt:
                pltpu.sync_copy(zeros16_h, cbuf)
                for k in range(5):
                    pltpu.sync_copy(cbuf, cnt_sh.at[pl.ds(sid * 640 + k * 128, 128)])
            plsc.subcore_barrier()

            pltpu.sync_copy(s4.at[q, sid], src_v)

            def chunk_body(j, carry):
                pltpu.async_copy(table.at[src_v.at[j]], g0, gsem).wait()
                pltpu.sync_copy(g0, acc_sh.at[dst_v.at[j]], add=True)
                if do_cnt:
                    @pl.when(j // (CPT // 2) == cid)
                    def _():
                        pltpu.sync_copy(ones_v, cnt_sh.at[dst_v.at[j]],
                                        add=True)
                return carry

            lax.fori_loop(0, CPT, chunk_body, 0)
            plsc.subcore_barrier()

            for k in range(5):
                r0 = sid * 640 + k * 128
                pltpu.sync_copy(acc_sh.at[pl.ds(r0, 128)], g0)
                pltpu.sync_copy(g0, agg_o.at[q, pl.ds(r0, 128)])

            if do_cnt:
                for k in range(5):
                    r0 = sid * 640 + k * 128
                    pltpu.sync_copy(cnt_sh.at[pl.ds(r0, 128)], cbuf)
                    pltpu.sync_copy(cbuf, cnt_o.at[cid, pl.ds(r0, 128)])

            plsc.subcore_barrier()


@jax.jit
def _sc_segment_sums(xp4, xa4, s_c, d_c, s_w, d_w, s_wb, d_wb):
    mesh = plsc.VectorSubcoreMesh(core_axis_name="c", subcore_axis_name="s")
    f32 = jnp.float32
    out_type = (
        jax.ShapeDtypeStruct((NQ, ACC_ROWS, DQ), f32),
        jax.ShapeDtypeStruct((NC, ACC_ROWS, 16), f32),
        jax.ShapeDtypeStruct((NQ, ACC_ROWS, DQ), f32),
        jax.ShapeDtypeStruct((NC, ACC_ROWS, 16), f32),
        jax.ShapeDtypeStruct((NQ, ACC_ROWS, DQ), f32),
        jax.ShapeDtypeStruct((NC, ACC_ROWS, 16), f32),
    )
    scratch = [
        pltpu.VMEM((CPT, CHUNK), jnp.int32),
        pltpu.VMEM((CPT, CHUNK), jnp.int32),
        pltpu.VMEM((CHUNK, DQ), f32),
        pltpu.VMEM((CHUNK, 16), f32),
        pltpu.VMEM((CHUNK, DQ), f32),
        pltpu.VMEM((CHUNK, 16), f32),
        pltpu.VMEM_SHARED((ACC_ROWS, DQ), f32),
        pltpu.VMEM_SHARED((ACC_ROWS, 16), f32),
        pltpu.SemaphoreType.DMA,
        pltpu.SemaphoreType.DMA,
        pltpu.SemaphoreType.DMA,
    ]
    zeros_h = jnp.zeros((CHUNK, DQ), f32)
    zeros16_h = jnp.zeros((CHUNK, 16), f32)
    ones_h = jnp.ones((CHUNK, 16), f32)
    return pl.kernel(_sc_body, out_type=out_type, mesh=mesh,
                     scratch_types=scratch,
                     compiler_params=pltpu.CompilerParams(
                         use_tc_tiling_on_sc=False))(
        xp4, xa4, s_c, d_c, s_w, d_w, s_wb, d_wb, zeros_h, zeros16_h, ones_h)


def _prep_edges(edge_index):
    e = edge_index.astype(jnp.int32)
    src = e[0].reshape(NS, E // NS)
    dst = e[1].reshape(NS, E // NS)
    pad = EPT - E // NS
    src = jnp.pad(src, ((0, 0), (0, pad))).reshape(NS, CPT, CHUNK)
    dst = jnp.pad(dst, ((0, 0), (0, pad)), constant_values=N).reshape(NS, CPT, CHUNK)
    src4 = jnp.stack([4 * src + q for q in range(NQ)])
    return src4, dst


R = 1024


def _tc_main_body(xp, xa, aggc, cntc, aggw, cntw, aggwb, cntwb,
                  Wrc, Wnc, bc, Wrw, Wnw, bw, Wrwb, Wnwb, bwb, Wkp, bkp, qp,
                  oc_ref, ow_ref, owb_ref, s_ref):
    i = pl.program_id(0)

    def conv(xd, agg, cnt, Wr, Wn, b):
        c = jnp.maximum(cnt[0, :, 0:1] + cnt[1, :, 0:1], 1.0)
        acc = xd @ Wr[...]
        for q in range(NQ):
            acc += (agg[q] / c) @ Wn[q * DQ:(q + 1) * DQ, :]
        return acc + b[...]

    xpb = xp[...]
    oc = conv(xpb, aggc[...], cntc[...], Wrc, Wnc, bc)
    ow = conv(xpb, aggw[...], cntw[...], Wrw, Wnw, bw)
    owb = conv(xa[...], aggwb[...], cntwb[...], Wrwb, Wnwb, bwb)
    oc_ref[...] = oc
    ow_ref[...] = ow
    owb_ref[...] = owb

    mask = (i * R + lax.broadcasted_iota(jnp.int32, (R, 1), 0)) < N

    def score(o):
        k = o @ Wkp[...] + bkp[...]
        k = jnp.where(k >= 0, k, 0.01 * k)
        return jnp.sum(jnp.where(mask, k * qp[...], 0.0))

    p = jnp.stack([score(oc), score(ow)])
    part = p[:, None] * jnp.ones((1, 128), jnp.float32)

    @pl.when(i == 0)
    def _():
        s_ref[...] = jnp.zeros_like(s_ref)

    s_ref[...] += part


def _tc_combine_body(oc, ow, s_ref, out_ref):
    s = s_ref[...] / N
    s0 = s[0, 0]
    s1 = s[1, 0]
    m = jnp.maximum(s0, s1)
    e0 = jnp.exp(s0 - m)
    e1 = jnp.exp(s1 - m)
    inv = 1.0 / (e0 + e1)
    out_ref[...] = (e0 * inv) * oc[...] + (e1 * inv) * ow[...]


def kernel(x_paper, x_author, edge_index_cites, edge_index_writes,
           edge_index_written_by, Wr_cites, Wn_cites, b_cites, Wr_writes,
           Wn_writes, b_writes, Wr_wb, Wn_wb, b_wb, Wk_paper, bk_paper,
           q_paper, Wk_author, bk_author, q_author):
    xp4 = x_paper.reshape(NQ * N, DQ)
    xa4 = x_author.reshape(NQ * N, DQ)
    s_c, d_c = _prep_edges(edge_index_cites)
    s_w, d_w = _prep_edges(edge_index_writes)
    s_wb, d_wb = _prep_edges(edge_index_written_by)

    agg_c, cnt_c, agg_w, cnt_w, agg_wb, cnt_wb = _sc_segment_sums(
        xp4, xa4, s_c, d_c, s_w, d_w, s_wb, d_wb)

    f32 = jnp.float32
    grid = ((N + R - 1) // R,)
    row = lambda i: (i, 0)
    row3 = lambda i: (0, i, 0)
    full = lambda i: (0, 0)
    cspec = pl.BlockSpec((NC, R, 16), row3)
    in_specs = [
        pl.BlockSpec((R, D), row),
        pl.BlockSpec((R, D), row),
        pl.BlockSpec((NQ, R, DQ), row3), cspec,
        pl.BlockSpec((NQ, R, DQ), row3), cspec,
        pl.BlockSpec((NQ, R, DQ), row3), cspec,
    ] + [pl.BlockSpec((D, D), full), pl.BlockSpec((D, D), full),
         pl.BlockSpec((1, D), full)] * 3 + [
        pl.BlockSpec((D, D), full),
        pl.BlockSpec((1, D), full),
        pl.BlockSpec((1, D), full),
    ]
    out_specs = [
        pl.BlockSpec((R, D), row), pl.BlockSpec((R, D), row),
        pl.BlockSpec((R, D), row), pl.BlockSpec((2, 128), full),
    ]
    oc, ow, out_author, s = pl.pallas_call(
        _tc_main_body, grid=grid, in_specs=in_specs, out_specs=out_specs,
        out_shape=[jax.ShapeDtypeStruct((N, D), f32)] * 3
        + [jax.ShapeDtypeStruct((2, 128), f32)],
    )(x_paper, x_author, agg_c, cnt_c, agg_w, cnt_w, agg_wb, cnt_wb,
      Wr_cites, Wn_cites, b_cites.reshape(1, D),
      Wr_writes, Wn_writes, b_writes.reshape(1, D),
      Wr_wb, Wn_wb, b_wb.reshape(1, D),
      Wk_paper, bk_paper.reshape(1, D), q_paper)

    out_paper = pl.pallas_call(
        _tc_combine_body, grid=grid,
        in_specs=[pl.BlockSpec((R, D), row), pl.BlockSpec((R, D), row),
                  pl.BlockSpec((2, 128), full)],
        out_specs=pl.BlockSpec((R, D), row),
        out_shape=jax.ShapeDtypeStruct((N, D), f32),
    )(oc, ow, s)

    return (out_paper, out_author)


# speedup vs baseline: 1.4125x; 1.4125x over previous
import jax
import jax.numpy as jnp
from jax import lax
from jax.experimental import pallas as pl
from jax.experimental.pallas import tpu as pltpu
from jax.experimental.pallas import tpu_sc as plsc

N = 10000
E = 160000
D = 256
DQ = 64
NQ = 4
NC = 2
NS = 16
CHUNK = 128
CPT = 79
EPT = CPT * CHUNK
ACC_ROWS = 10240


def _sc_body(xp4, xa4, s_c, d_c, s_w, d_w, s_wb, d_wb, zeros_h, zeros16_h,
             ones_h,
             agg_c, cnt_c, agg_w, cnt_w, agg_wb, cnt_wb,
             src_v, dst_v, gbuf, ones_v, zer_v, cbuf, acc_sh, cnt_sh, sem):
    cid = lax.axis_index("c")
    sid = lax.axis_index("s")
    pltpu.sync_copy(zeros_h, zer_v)
    pltpu.sync_copy(ones_h, ones_v)

    for table, s4, d3, agg_o, cnt_o in (
        (xp4, s_c, d_c, agg_c, cnt_c),
        (xa4, s_w, d_w, agg_w, cnt_w),
        (xp4, s_wb, d_wb, agg_wb, cnt_wb),
    ):
        pltpu.sync_copy(d3.at[sid], dst_v)
        for p in range(2):
            q = 2 * p + cid
            do_cnt = p == 0
            for k in range(5):
                pltpu.sync_copy(zer_v, acc_sh.at[pl.ds(sid * 640 + k * 128, 128)])
            if do_cnt:
                @pl.when(cid == 0)
                def _():
                    pltpu.sync_copy(zeros16_h, cbuf)
                    for k in range(5):
                        pltpu.sync_copy(cbuf, cnt_sh.at[pl.ds(sid * 640 + k * 128, 128)])
            plsc.subcore_barrier()

            pltpu.sync_copy(s4.at[q, sid], src_v)

            if p == 0:
                def chunk_body(j, carry):
                    pltpu.async_copy(table.at[src_v.at[j]], gbuf, sem).wait()
                    pltpu.sync_copy(gbuf, acc_sh.at[dst_v.at[j]], add=True)

                    @pl.when(cid == 0)
                    def _():
                        pltpu.sync_copy(ones_v, cnt_sh.at[dst_v.at[j]], add=True)

                    return carry
            else:
                def chunk_body(j, carry):
                    pltpu.async_copy(table.at[src_v.at[j]], gbuf, sem).wait()
                    pltpu.sync_copy(gbuf, acc_sh.at[dst_v.at[j]], add=True)
                    return carry

            lax.fori_loop(0, CPT, chunk_body, 0)
            plsc.subcore_barrier()

            for k in range(5):
                r0 = sid * 640 + k * 128
                pltpu.sync_copy(acc_sh.at[pl.ds(r0, 128)], gbuf)
                pltpu.sync_copy(gbuf, agg_o.at[q, pl.ds(r0, 128)])

            if do_cnt:
                @pl.when(cid == 0)
                def _():
                    for k in range(5):
                        r0 = sid * 640 + k * 128
                        pltpu.sync_copy(cnt_sh.at[pl.ds(r0, 128)], cbuf)
                        pltpu.sync_copy(cbuf, cnt_o.at[pl.ds(r0, 128)])

            plsc.subcore_barrier()


@jax.jit
def _sc_segment_sums(xp4, xa4, s_c, d_c, s_w, d_w, s_wb, d_wb):
    mesh = plsc.VectorSubcoreMesh(core_axis_name="c", subcore_axis_name="s")
    f32 = jnp.float32
    out_type = (
        jax.ShapeDtypeStruct((NQ, ACC_ROWS, DQ), f32),
        jax.ShapeDtypeStruct((ACC_ROWS, 16), f32),
        jax.ShapeDtypeStruct((NQ, ACC_ROWS, DQ), f32),
        jax.ShapeDtypeStruct((ACC_ROWS, 16), f32),
        jax.ShapeDtypeStruct((NQ, ACC_ROWS, DQ), f32),
        jax.ShapeDtypeStruct((ACC_ROWS, 16), f32),
    )
    scratch = [
        pltpu.VMEM((CPT, CHUNK), jnp.int32),
        pltpu.VMEM((CPT, CHUNK), jnp.int32),
        pltpu.VMEM((CHUNK, DQ), f32),
        pltpu.VMEM((CHUNK, 16), f32),
        pltpu.VMEM((CHUNK, DQ), f32),
        pltpu.VMEM((CHUNK, 16), f32),
        pltpu.VMEM_SHARED((ACC_ROWS, DQ), f32),
        pltpu.VMEM_SHARED((ACC_ROWS, 16), f32),
        pltpu.SemaphoreType.DMA,
    ]
    zeros_h = jnp.zeros((CHUNK, DQ), f32)
    zeros16_h = jnp.zeros((CHUNK, 16), f32)
    ones_h = jnp.ones((CHUNK, 16), f32)
    return pl.kernel(_sc_body, out_type=out_type, mesh=mesh,
                     scratch_types=scratch,
                     compiler_params=pltpu.CompilerParams(
                         use_tc_tiling_on_sc=False))(
        xp4, xa4, s_c, d_c, s_w, d_w, s_wb, d_wb, zeros_h, zeros16_h, ones_h)


def _prep_edges(edge_index):
    e = edge_index.astype(jnp.int32)
    src = e[0].reshape(NS, E // NS)
    dst = e[1].reshape(NS, E // NS)
    pad = EPT - E // NS
    src = jnp.pad(src, ((0, 0), (0, pad))).reshape(NS, CPT, CHUNK)
    dst = jnp.pad(dst, ((0, 0), (0, pad)), constant_values=N).reshape(NS, CPT, CHUNK)
    src4 = jnp.stack([4 * src + q for q in range(NQ)])
    return src4, dst


R = 1024


def _tc_main_body(xp, xa, aggc, cntc, aggw, cntw, aggwb, cntwb,
                  Wrc, Wnc, bc, Wrw, Wnw, bw, Wrwb, Wnwb, bwb, Wkp, bkp, qp,
                  oc_ref, ow_ref, owb_ref, s_ref):
    i = pl.program_id(0)

    def conv(xd, agg, cnt, Wr, Wn, b):
        c = jnp.maximum(cnt[:, 0:1], 1.0)
        acc = xd @ Wr[...]
        for q in range(NQ):
            acc += (agg[q] / c) @ Wn[q * DQ:(q + 1) * DQ, :]
        return acc + b[...]

    xpb = xp[...]
    oc = conv(xpb, aggc[...], cntc[...], Wrc, Wnc, bc)
    ow = conv(xpb, aggw[...], cntw[...], Wrw, Wnw, bw)
    owb = conv(xa[...], aggwb[...], cntwb[...], Wrwb, Wnwb, bwb)
    oc_ref[...] = oc
    ow_ref[...] = ow
    owb_ref[...] = owb

    mask = (i * R + lax.broadcasted_iota(jnp.int32, (R, 1), 0)) < N

    def score(o):
        k = o @ Wkp[...] + bkp[...]
        k = jnp.where(k >= 0, k, 0.01 * k)
        return jnp.sum(jnp.where(mask, k * qp[...], 0.0))

    p = jnp.stack([score(oc), score(ow)])
    part = p[:, None] * jnp.ones((1, 128), jnp.float32)

    @pl.when(i == 0)
    def _():
        s_ref[...] = jnp.zeros_like(s_ref)

    s_ref[...] += part


def _tc_combine_body(oc, ow, s_ref, out_ref):
    s = s_ref[...] / N
    s0 = s[0, 0]
    s1 = s[1, 0]
    m = jnp.maximum(s0, s1)
    e0 = jnp.exp(s0 - m)
    e1 = jnp.exp(s1 - m)
    inv = 1.0 / (e0 + e1)
    out_ref[...] = (e0 * inv) * oc[...] + (e1 * inv) * ow[...]


def kernel(x_paper, x_author, edge_index_cites, edge_index_writes,
           edge_index_written_by, Wr_cites, Wn_cites, b_cites, Wr_writes,
           Wn_writes, b_writes, Wr_wb, Wn_wb, b_wb, Wk_paper, bk_paper,
           q_paper, Wk_author, bk_author, q_author):
    xp4 = x_paper.reshape(NQ * N, DQ)
    xa4 = x_author.reshape(NQ * N, DQ)
    s_c, d_c = _prep_edges(edge_index_cites)
    s_w, d_w = _prep_edges(edge_index_writes)
    s_wb, d_wb = _prep_edges(edge_index_written_by)

    agg_c, cnt_c, agg_w, cnt_w, agg_wb, cnt_wb = _sc_segment_sums(
        xp4, xa4, s_c, d_c, s_w, d_w, s_wb, d_wb)

    f32 = jnp.float32
    grid = ((N + R - 1) // R,)
    row = lambda i: (i, 0)
    row3 = lambda i: (0, i, 0)
    full = lambda i: (0, 0)
    cspec = pl.BlockSpec((R, 16), row)
    in_specs = [
        pl.BlockSpec((R, D), row),
        pl.BlockSpec((R, D), row),
        pl.BlockSpec((NQ, R, DQ), row3), cspec,
        pl.BlockSpec((NQ, R, DQ), row3), cspec,
        pl.BlockSpec((NQ, R, DQ), row3), cspec,
    ] + [pl.BlockSpec((D, D), full), pl.BlockSpec((D, D), full),
         pl.BlockSpec((1, D), full)] * 3 + [
        pl.BlockSpec((D, D), full),
        pl.BlockSpec((1, D), full),
        pl.BlockSpec((1, D), full),
    ]
    out_specs = [
        pl.BlockSpec((R, D), row), pl.BlockSpec((R, D), row),
        pl.BlockSpec((R, D), row), pl.BlockSpec((2, 128), full),
    ]
    oc, ow, out_author, s = pl.pallas_call(
        _tc_main_body, grid=grid, in_specs=in_specs, out_specs=out_specs,
        out_shape=[jax.ShapeDtypeStruct((N, D), f32)] * 3
        + [jax.ShapeDtypeStruct((2, 128), f32)],
    )(x_paper, x_author, agg_c, cnt_c, agg_w, cnt_w, agg_wb, cnt_wb,
      Wr_cites, Wn_cites, b_cites.reshape(1, D),
      Wr_writes, Wn_writes, b_writes.reshape(1, D),
      Wr_wb, Wn_wb, b_wb.reshape(1, D),
      Wk_paper, bk_paper.reshape(1, D), q_paper)

    out_paper = pl.pallas_call(
        _tc_combine_body, grid=grid,
        in_specs=[pl.BlockSpec((R, D), row), pl.BlockSpec((R, D), row),
                  pl.BlockSpec((2, 128), full)],
        out_specs=pl.BlockSpec((R, D), row),
        out_shape=jax.ShapeDtypeStruct((N, D), f32),
    )(oc, ow, s)

    return (out_paper, out_author)


# speedup vs baseline: 1.4175x; 1.0036x over previous
import jax
import jax.numpy as jnp
from jax import lax
from jax.experimental import pallas as pl
from jax.experimental.pallas import tpu as pltpu
from jax.experimental.pallas import tpu_sc as plsc

N = 10000
E = 160000
D = 256
DQ = 64
NQ = 4
NC = 2
NS = 16
CHUNK = 128
CPT = 79
EPT = CPT * CHUNK
ACC_ROWS = 10240


def _sc_body(xp4, xa4, s_c, d_c, s_w, d_w, s_wb, d_wb, zeros_h, zeros16_h,
             ones_h,
             agg_c, cnt_c, agg_w, cnt_w, agg_wb, cnt_wb,
             src_v, dst_v, gbuf, ones_v, zer_v, cbuf, acc_sh, cnt_sh, sem):
    cid = lax.axis_index("c")
    sid = lax.axis_index("s")
    pltpu.sync_copy(zeros_h, zer_v)
    pltpu.sync_copy(ones_h, ones_v)

    for table, s4, d3, agg_o, cnt_o, cnt_p, cnt_c in (
        (xp4, s_c, d_c, agg_c, cnt_c, 0, 0),
        (xa4, s_w, d_w, agg_w, cnt_w, 0, 1),
        (xp4, s_wb, d_wb, agg_wb, cnt_wb, 1, 1),
    ):
        pltpu.sync_copy(d3.at[sid], dst_v)
        for p in range(2):
            q = 2 * p + cid
            do_cnt = p == cnt_p
            for k in range(5):
                pltpu.sync_copy(zer_v, acc_sh.at[pl.ds(sid * 640 + k * 128, 128)])
            if do_cnt:
                @pl.when(cid == cnt_c)
                def _():
                    pltpu.sync_copy(zeros16_h, cbuf)
                    for k in range(5):
                        pltpu.sync_copy(cbuf, cnt_sh.at[pl.ds(sid * 640 + k * 128, 128)])
            plsc.subcore_barrier()

            pltpu.sync_copy(s4.at[q, sid], src_v)

            if do_cnt:
                def chunk_body(j, carry):
                    pltpu.async_copy(table.at[src_v.at[j]], gbuf, sem).wait()
                    pltpu.sync_copy(gbuf, acc_sh.at[dst_v.at[j]], add=True)

                    @pl.when(cid == cnt_c)
                    def _():
                        pltpu.sync_copy(ones_v, cnt_sh.at[dst_v.at[j]], add=True)

                    return carry
            else:
                def chunk_body(j, carry):
                    pltpu.async_copy(table.at[src_v.at[j]], gbuf, sem).wait()
                    pltpu.sync_copy(gbuf, acc_sh.at[dst_v.at[j]], add=True)
                    return carry

            lax.fori_loop(0, CPT, chunk_body, 0)
            plsc.subcore_barrier()

            for k in range(5):
                r0 = sid * 640 + k * 128
                pltpu.sync_copy(acc_sh.at[pl.ds(r0, 128)], gbuf)
                pltpu.sync_copy(gbuf, agg_o.at[q, pl.ds(r0, 128)])

            if do_cnt:
                @pl.when(cid == cnt_c)
                def _():
                    for k in range(5):
                        r0 = sid * 640 + k * 128
                        pltpu.sync_copy(cnt_sh.at[pl.ds(r0, 128)], cbuf)
                        pltpu.sync_copy(cbuf, cnt_o.at[pl.ds(r0, 128)])

            plsc.subcore_barrier()


@jax.jit
def _sc_segment_sums(xp4, xa4, s_c, d_c, s_w, d_w, s_wb, d_wb):
    mesh = plsc.VectorSubcoreMesh(core_axis_name="c", subcore_axis_name="s")
    f32 = jnp.float32
    out_type = (
        jax.ShapeDtypeStruct((NQ, ACC_ROWS, DQ), f32),
        jax.ShapeDtypeStruct((ACC_ROWS, 16), f32),
        jax.ShapeDtypeStruct((NQ, ACC_ROWS, DQ), f32),
        jax.ShapeDtypeStruct((ACC_ROWS, 16), f32),
        jax.ShapeDtypeStruct((NQ, ACC_ROWS, DQ), f32),
        jax.ShapeDtypeStruct((ACC_ROWS, 16), f32),
    )
    scratch = [
        pltpu.VMEM((CPT, CHUNK), jnp.int32),
        pltpu.VMEM((CPT, CHUNK), jnp.int32),
        pltpu.VMEM((CHUNK, DQ), f32),
        pltpu.VMEM((CHUNK, 16), f32),
        pltpu.VMEM((CHUNK, DQ), f32),
        pltpu.VMEM((CHUNK, 16), f32),
        pltpu.VMEM_SHARED((ACC_ROWS, DQ), f32),
        pltpu.VMEM_SHARED((ACC_ROWS, 16), f32),
        pltpu.SemaphoreType.DMA,
    ]
    zeros_h = jnp.zeros((CHUNK, DQ), f32)
    zeros16_h = jnp.zeros((CHUNK, 16), f32)
    ones_h = jnp.ones((CHUNK, 16), f32)
    return pl.kernel(_sc_body, out_type=out_type, mesh=mesh,
                     scratch_types=scratch,
                     compiler_params=pltpu.CompilerParams(
                         use_tc_tiling_on_sc=False))(
        xp4, xa4, s_c, d_c, s_w, d_w, s_wb, d_wb, zeros_h, zeros16_h, ones_h)


def _prep_edges(edge_index):
    e = edge_index.astype(jnp.int32)
    src = e[0].reshape(NS, E // NS)
    dst = e[1].reshape(NS, E // NS)
    pad = EPT - E // NS
    src = jnp.pad(src, ((0, 0), (0, pad))).reshape(NS, CPT, CHUNK)
    dst = jnp.pad(dst, ((0, 0), (0, pad)), constant_values=N).reshape(NS, CPT, CHUNK)
    src4 = jnp.stack([4 * src + q for q in range(NQ)])
    return src4, dst


R = 1024


def _tc_main_body(xp, xa, aggc, cntc, aggw, cntw, aggwb, cntwb,
                  Wrc, Wnc, bc, Wrw, Wnw, bw, Wrwb, Wnwb, bwb, Wkp, bkp, qp,
                  oc_ref, ow_ref, owb_ref, s_ref):
    i = pl.program_id(0)

    def conv(xd, agg, cnt, Wr, Wn, b):
        c = jnp.maximum(cnt[:, 0:1], 1.0)
        acc = xd @ Wr[...]
        for q in range(NQ):
            acc += (agg[q] / c) @ Wn[q * DQ:(q + 1) * DQ, :]
        return acc + b[...]

    xpb = xp[...]
    oc = conv(xpb, aggc[...], cntc[...], Wrc, Wnc, bc)
    ow = conv(xpb, aggw[...], cntw[...], Wrw, Wnw, bw)
    owb = conv(xa[...], aggwb[...], cntwb[...], Wrwb, Wnwb, bwb)
    oc_ref[...] = oc
    ow_ref[...] = ow
    owb_ref[...] = owb

    mask = (i * R + lax.broadcasted_iota(jnp.int32, (R, 1), 0)) < N

    def score(o):
        k = o @ Wkp[...] + bkp[...]
        k = jnp.where(k >= 0, k, 0.01 * k)
        return jnp.sum(jnp.where(mask, k * qp[...], 0.0))

    p = jnp.stack([score(oc), score(ow)])
    part = p[:, None] * jnp.ones((1, 128), jnp.float32)

    @pl.when(i == 0)
    def _():
        s_ref[...] = jnp.zeros_like(s_ref)

    s_ref[...] += part


def _tc_combine_body(oc, ow, s_ref, out_ref):
    s = s_ref[...] / N
    s0 = s[0, 0]
    s1 = s[1, 0]
    m = jnp.maximum(s0, s1)
    e0 = jnp.exp(s0 - m)
    e1 = jnp.exp(s1 - m)
    inv = 1.0 / (e0 + e1)
    out_ref[...] = (e0 * inv) * oc[...] + (e1 * inv) * ow[...]


def kernel(x_paper, x_author, edge_index_cites, edge_index_writes,
           edge_index_written_by, Wr_cites, Wn_cites, b_cites, Wr_writes,
           Wn_writes, b_writes, Wr_wb, Wn_wb, b_wb, Wk_paper, bk_paper,
           q_paper, Wk_author, bk_author, q_author):
    xp4 = x_paper.reshape(NQ * N, DQ)
    xa4 = x_author.reshape(NQ * N, DQ)
    s_c, d_c = _prep_edges(edge_index_cites)
    s_w, d_w = _prep_edges(edge_index_writes)
    s_wb, d_wb = _prep_edges(edge_index_written_by)

    agg_c, cnt_c, agg_w, cnt_w, agg_wb, cnt_wb = _sc_segment_sums(
        xp4, xa4, s_c, d_c, s_w, d_w, s_wb, d_wb)

    f32 = jnp.float32
    grid = ((N + R - 1) // R,)
    row = lambda i: (i, 0)
    row3 = lambda i: (0, i, 0)
    full = lambda i: (0, 0)
    cspec = pl.BlockSpec((R, 16), row)
    in_specs = [
        pl.BlockSpec((R, D), row),
        pl.BlockSpec((R, D), row),
        pl.BlockSpec((NQ, R, DQ), row3), cspec,
        pl.BlockSpec((NQ, R, DQ), row3), cspec,
        pl.BlockSpec((NQ, R, DQ), row3), cspec,
    ] + [pl.BlockSpec((D, D), full), pl.BlockSpec((D, D), full),
         pl.BlockSpec((1, D), full)] * 3 + [
        pl.BlockSpec((D, D), full),
        pl.BlockSpec((1, D), full),
        pl.BlockSpec((1, D), full),
    ]
    out_specs = [
        pl.BlockSpec((R, D), row), pl.BlockSpec((R, D), row),
        pl.BlockSpec((R, D), row), pl.BlockSpec((2, 128), full),
    ]
    oc, ow, out_author, s = pl.pallas_call(
        _tc_main_body, grid=grid, in_specs=in_specs, out_specs=out_specs,
        out_shape=[jax.ShapeDtypeStruct((N, D), f32)] * 3
        + [jax.ShapeDtypeStruct((2, 128), f32)],
    )(x_paper, x_author, agg_c, cnt_c, agg_w, cnt_w, agg_wb, cnt_wb,
      Wr_cites, Wn_cites, b_cites.reshape(1, D),
      Wr_writes, Wn_writes, b_writes.reshape(1, D),
      Wr_wb, Wn_wb, b_wb.reshape(1, D),
      Wk_paper, bk_paper.reshape(1, D), q_paper)

    out_paper = pl.pallas_call(
        _tc_combine_body, grid=grid,
        in_specs=[pl.BlockSpec((R, D), row), pl.BlockSpec((R, D), row),
                  pl.BlockSpec((2, 128), full)],
        out_specs=pl.BlockSpec((R, D), row),
        out_shape=jax.ShapeDtypeStruct((N, D), f32),
    )(oc, ow, s)

    return (out_paper, out_author)
